# Initial kernel scaffold; baseline (speedup 1.0000x reference)
import functools

import jax
import jax.numpy as jnp
from jax import lax
from jax.experimental import pallas as pl
from jax.experimental.pallas import tpu as pltpu, tpu_sc as plsc

N_NODES = 10000
N_EDGES = 320000
D_FEAT = 128
D_EDGE = 16

NW = 32
EP = N_EDGES // NW
SUB = 80
CB = 400
NSUB = CB // SUB
NCH = EP // CB
NG = CB // 16
ZR = N_NODES // 16


def _nodeproj_body(af_ref, w1_ref, w2_ref, wab_ref, a_ref, b_ref, ab_ref):
    x = af_ref[...]
    a_ref[...] = jnp.dot(x, w1_ref[...], preferred_element_type=jnp.float32)
    b_ref[...] = jnp.dot(x, w2_ref[...], preferred_element_type=jnp.float32)
    ab_ref[...] = jnp.dot(x, wab_ref[...], preferred_element_type=jnp.float32)


def _node_projections(af, w1, w2, wab):
    blk = 2000
    grid = (N_NODES // blk,)
    return pl.pallas_call(
        _nodeproj_body,
        grid=grid,
        in_specs=[
            pl.BlockSpec((blk, D_FEAT), lambda i: (i, 0)),
            pl.BlockSpec((D_FEAT, D_EDGE), lambda i: (0, 0)),
            pl.BlockSpec((D_FEAT, D_EDGE), lambda i: (0, 0)),
            pl.BlockSpec((D_FEAT, 2), lambda i: (0, 0)),
        ],
        out_specs=[
            pl.BlockSpec((blk, D_EDGE), lambda i: (i, 0)),
            pl.BlockSpec((blk, D_EDGE), lambda i: (i, 0)),
            pl.BlockSpec((blk, 2), lambda i: (i, 0)),
        ],
        out_shape=[
            jax.ShapeDtypeStruct((N_NODES, D_EDGE), jnp.float32),
            jax.ShapeDtypeStruct((N_NODES, D_EDGE), jnp.float32),
            jax.ShapeDtypeStruct((N_NODES, 2), jnp.float32),
        ],
    )(af, w1, w2, wab)


def _edgeproj_body(bf_ref, w3_ref, be_ref, w4_ref, ba_ref, c_ref, ca_ref):
    x = bf_ref[...]
    c_ref[...] = jnp.dot(x, w3_ref[...], preferred_element_type=jnp.float32) + be_ref[...]
    ca_ref[...] = jnp.dot(x, w4_ref[...], preferred_element_type=jnp.float32) + ba_ref[...]


def _edge_projections(bf, w3, be, w4, ba):
    blk = 8000
    grid = (N_EDGES // blk,)
    return pl.pallas_call(
        _edgeproj_body,
        grid=grid,
        in_specs=[
            pl.BlockSpec((blk, D_EDGE), lambda i: (i, 0)),
            pl.BlockSpec((D_EDGE, D_EDGE), lambda i: (0, 0)),
            pl.BlockSpec((1, D_EDGE), lambda i: (0, 0)),
            pl.BlockSpec((D_EDGE, 1), lambda i: (0, 0)),
            pl.BlockSpec((1, 1), lambda i: (0, 0)),
        ],
        out_specs=[
            pl.BlockSpec((blk, D_EDGE), lambda i: (i, 0)),
            pl.BlockSpec((blk, 1), lambda i: (i, 0)),
        ],
        out_shape=[
            jax.ShapeDtypeStruct((N_EDGES, D_EDGE), jnp.float32),
            jax.ShapeDtypeStruct((N_EDGES, 1), jnp.float32),
        ],
    )(bf, w3, be, w4, ba)


def _sc_edge_body(a_h, b_h, an_h, bn_h, src2_h, dst2_h, c_h, ca_h,
                  ubf_h, part_h,
                  an_v, bn_v, idxs_v, idxd_v, rowsa_v, rowsb_v,
                  c_v, ca_v, ubf_v, msg_v, agg_sh, sem):
    c_id = lax.axis_index("c")
    s_id = lax.axis_index("s")
    wid = s_id * 2 + c_id

    pltpu.sync_copy(an_h, an_v)
    pltpu.sync_copy(bn_h, bn_v)

    zeros16 = jnp.zeros((16,), jnp.float32)

    def _zrow(i, carry):
        msg_v[i, :] = zeros16
        return carry

    lax.fori_loop(0, CB, _zrow, 0)
    zbase = s_id * ZR
    pltpu.sync_copy(msg_v, agg_sh.at[pl.ds(zbase, CB)])
    pltpu.sync_copy(msg_v.at[pl.ds(0, ZR - CB), :],
                    agg_sh.at[pl.ds(zbase + CB, ZR - CB)])
    plsc.subcore_barrier()

    def _chunk(ci, carry):
        ebase = wid * EP + ci * CB
        rbase = wid * (EP // SUB) + ci * NSUB
        pltpu.sync_copy(src2_h.at[pl.ds(rbase, NSUB)], idxs_v)
        pltpu.sync_copy(dst2_h.at[pl.ds(rbase, NSUB)], idxd_v)
        pltpu.sync_copy(c_h.at[pl.ds(ebase, CB)], c_v)
        pltpu.sync_copy(ca_h.at[pl.ds(ebase, CB)], ca_v)
        cps = []
        for j in range(NSUB):
            cps.append(pltpu.async_copy(
                a_h.at[idxs_v.at[j]], rowsa_v.at[pl.ds(j * SUB, SUB)], sem))
            cps.append(pltpu.async_copy(
                b_h.at[idxd_v.at[j]], rowsb_v.at[pl.ds(j * SUB, SUB)], sem))
        for cp in cps:
            cp.wait()

        def _group(g, carry2):
            r = g // NSUB
            q = g - r * NSUB
            srcv = idxs_v[r, pl.ds(q * 16, 16)]
            dstv = idxd_v[r, pl.ds(q * 16, 16)]
            av = plsc.load_gather(an_v, [srcv])
            bv = plsc.load_gather(bn_v, [dstv])
            t = av + bv + ca_v[pl.ds(g * 16, 16)]
            sig = 1.0 / (1.0 + jnp.exp(-t))
            for e in range(16):
                ee = g * 16 + e
                u = jnp.maximum(rowsa_v[ee, :] + rowsb_v[ee, :] + c_v[ee, :], 0.0)
                ubf_v[ee, :] = u
                msg_v[ee, :] = u * sig[e]
            return carry2

        lax.fori_loop(0, NG, _group, 0)
        pltpu.sync_copy(ubf_v, ubf_h.at[pl.ds(ebase, CB)])
        for j in range(NSUB):
            pltpu.sync_copy(msg_v.at[pl.ds(j * SUB, SUB), :],
                            agg_sh.at[idxd_v.at[j]], add=True)
        return carry

    lax.fori_loop(0, NCH, _chunk, 0)
    plsc.subcore_barrier()

    pltpu.sync_copy(agg_sh.at[pl.ds(zbase, ZR)],
                    part_h.at[c_id, pl.ds(zbase, ZR)])


def _sc_edge(a_n, b_n, an, bn, src2, dst2, c_e, ca_e):
    mesh = plsc.VectorSubcoreMesh(core_axis_name="c", subcore_axis_name="s")
    kern = functools.partial(
        pl.kernel,
        mesh=mesh,
        out_type=[
            jax.ShapeDtypeStruct((N_EDGES, D_EDGE), jnp.float32),
            jax.ShapeDtypeStruct((2, N_NODES, D_EDGE), jnp.float32),
        ],
        scratch_types=[
            pltpu.VMEM((N_NODES,), jnp.float32),
            pltpu.VMEM((N_NODES,), jnp.float32),
            pltpu.VMEM((NSUB, SUB), jnp.int32),
            pltpu.VMEM((NSUB, SUB), jnp.int32),
            pltpu.VMEM((CB, D_EDGE), jnp.float32),
            pltpu.VMEM((CB, D_EDGE), jnp.float32),
            pltpu.VMEM((CB, D_EDGE), jnp.float32),
            pltpu.VMEM((CB,), jnp.float32),
            pltpu.VMEM((CB, D_EDGE), jnp.float32),
            pltpu.VMEM((CB, D_EDGE), jnp.float32),
            pltpu.VMEM_SHARED((N_NODES, D_EDGE), jnp.float32),
            pltpu.SemaphoreType.DMA,
        ],
    )(_sc_edge_body)
    return kern(a_n, b_n, an, bn, src2, dst2, c_e, ca_e)


def _nodemodel_body(af_ref, p0_ref, p1_ref, wn1_ref, wn2_ref, bn_ref, out_ref):
    acc = jnp.dot(af_ref[...], wn1_ref[...], preferred_element_type=jnp.float32)
    acc = acc + jnp.dot(p0_ref[...] + p1_ref[...], wn2_ref[...],
                        preferred_element_type=jnp.float32)
    out_ref[...] = jnp.maximum(acc + bn_ref[...], 0.0)


def _node_model(af, p0, p1, wn1, wn2, bn):
    blk = 2000
    grid = (N_NODES // blk,)
    return pl.pallas_call(
        _nodemodel_body,
        grid=grid,
        in_specs=[
            pl.BlockSpec((blk, D_FEAT), lambda i: (i, 0)),
            pl.BlockSpec((blk, D_EDGE), lambda i: (i, 0)),
            pl.BlockSpec((blk, D_EDGE), lambda i: (i, 0)),
            pl.BlockSpec((D_FEAT, D_FEAT), lambda i: (0, 0)),
            pl.BlockSpec((D_EDGE, D_FEAT), lambda i: (0, 0)),
            pl.BlockSpec((1, D_FEAT), lambda i: (0, 0)),
        ],
        out_specs=pl.BlockSpec((blk, D_FEAT), lambda i: (i, 0)),
        out_shape=jax.ShapeDtypeStruct((N_NODES, D_FEAT), jnp.float32),
    )(af, p0, p1, wn1, wn2, bn)


def kernel(af, edge_index, bf, W_edge, b_edge, W_attn, b_attn, W_node, b_node):
    w1 = W_edge[:D_FEAT]
    w2 = W_edge[D_FEAT:2 * D_FEAT]
    w3 = W_edge[2 * D_FEAT:]
    wab = jnp.concatenate([W_attn[:D_FEAT], W_attn[D_FEAT:2 * D_FEAT]], axis=1)
    w4 = W_attn[2 * D_FEAT:]

    a_n, b_n, ab_n = _node_projections(af, w1, w2, wab)
    c_e, ca_e = _edge_projections(bf, w3, b_edge.reshape(1, D_EDGE),
                                  w4, b_attn.reshape(1, 1))

    src2 = edge_index[0].reshape(N_EDGES // SUB, SUB)
    dst2 = edge_index[1].reshape(N_EDGES // SUB, SUB)
    an = ab_n[:, 0]
    bn = ab_n[:, 1]

    ubf, part = _sc_edge(a_n, b_n, an, bn, src2, dst2,
                         c_e, ca_e.reshape(N_EDGES))

    uaf = _node_model(af, part[0], part[1],
                      W_node[:D_FEAT], W_node[D_FEAT:],
                      b_node.reshape(1, D_FEAT))
    return (uaf, ubf)


# speedup vs baseline: 2.8886x; 2.8886x over previous
import functools

import jax
import jax.numpy as jnp
from jax import lax
from jax.experimental import pallas as pl
from jax.experimental.pallas import tpu as pltpu, tpu_sc as plsc

N_NODES = 10000
N_EDGES = 320000
D_FEAT = 128
D_EDGE = 16

NW = 32
EP = N_EDGES // NW
SUB = 80
CB = 400
NSUB = CB // SUB
NCH = EP // CB
NG = CB // 16
ZR = 624
ZTAIL = N_NODES - 16 * ZR


def _nodeproj_body(af_ref, w1_ref, w2_ref, a_ref, b_ref):
    x = af_ref[...]
    a_ref[...] = jnp.dot(x, w1_ref[...], preferred_element_type=jnp.float32)
    b_ref[...] = jnp.dot(x, w2_ref[...], preferred_element_type=jnp.float32)


def _node_projections(af, w1p, w2p):
    blk = 2000
    grid = (N_NODES // blk,)
    return pl.pallas_call(
        _nodeproj_body,
        grid=grid,
        in_specs=[
            pl.BlockSpec((blk, D_FEAT), lambda i: (i, 0)),
            pl.BlockSpec((D_FEAT, 32), lambda i: (0, 0)),
            pl.BlockSpec((D_FEAT, 32), lambda i: (0, 0)),
        ],
        out_specs=[
            pl.BlockSpec((blk, 32), lambda i: (i, 0)),
            pl.BlockSpec((blk, 32), lambda i: (i, 0)),
        ],
        out_shape=[
            jax.ShapeDtypeStruct((N_NODES, 32), jnp.float32),
            jax.ShapeDtypeStruct((N_NODES, 32), jnp.float32),
        ],
    )(af, w1p, w2p)


def _edgeproj_body(bf_ref, w3_ref, be_ref, w4_ref, ba_ref, c_ref, ca_ref):
    x = bf_ref[...]
    c_ref[...] = jnp.dot(x, w3_ref[...], preferred_element_type=jnp.float32) + be_ref[...]
    ca_ref[...] = jnp.dot(x, w4_ref[...], preferred_element_type=jnp.float32) + ba_ref[...]


def _edge_projections(bf, w3, be, w4, ba):
    blk = 8000
    grid = (N_EDGES // blk,)
    return pl.pallas_call(
        _edgeproj_body,
        grid=grid,
        in_specs=[
            pl.BlockSpec((blk, D_EDGE), lambda i: (i, 0)),
            pl.BlockSpec((D_EDGE, D_EDGE), lambda i: (0, 0)),
            pl.BlockSpec((1, D_EDGE), lambda i: (0, 0)),
            pl.BlockSpec((D_EDGE, 1), lambda i: (0, 0)),
            pl.BlockSpec((1, 1), lambda i: (0, 0)),
        ],
        out_specs=[
            pl.BlockSpec((blk, D_EDGE), lambda i: (i, 0)),
            pl.BlockSpec((blk, 1), lambda i: (i, 0)),
        ],
        out_shape=[
            jax.ShapeDtypeStruct((N_EDGES, D_EDGE), jnp.float32),
            jax.ShapeDtypeStruct((N_EDGES, 1), jnp.float32),
        ],
    )(bf, w3, be, w4, ba)


def _sc_edge_body(a_h, b_h, src_h, dst_h, c_h, ca_h,
                  ubf_h, part_h,
                  idxs_v, idxd_v, rowsa_v, rowsb_v,
                  c_v, ca_v, ubf_v, msg_v, agg_sh, sem):
    c_id = lax.axis_index("c")
    s_id = lax.axis_index("s")
    wid = s_id * 2 + c_id

    zeros16 = jnp.zeros((16,), jnp.float32)

    def _zrow(i, carry):
        msg_v[i, :] = zeros16
        return carry

    lax.fori_loop(0, CB, _zrow, 0)
    zbase = s_id * ZR
    pltpu.sync_copy(msg_v, agg_sh.at[pl.ds(zbase, CB)])
    pltpu.sync_copy(msg_v.at[pl.ds(0, ZR - CB), :],
                    agg_sh.at[pl.ds(zbase + CB, ZR - CB)])

    @pl.when(s_id == 15)
    def _ztail():
        pltpu.sync_copy(msg_v.at[pl.ds(0, ZTAIL), :],
                        agg_sh.at[pl.ds(16 * ZR, ZTAIL)])

    plsc.subcore_barrier()

    def _chunk(ci, carry):
        ebase = wid * EP + ci * CB
        for j in range(NSUB):
            pltpu.sync_copy(src_h.at[pl.ds(ebase + j * SUB, SUB)],
                            idxs_v.at[j])
            pltpu.sync_copy(dst_h.at[pl.ds(ebase + j * SUB, SUB)],
                            idxd_v.at[j])
        pltpu.sync_copy(c_h.at[pl.ds(ebase, CB)], c_v)
        pltpu.sync_copy(ca_h.at[pl.ds(ebase, CB)], ca_v)
        cps = []
        for j in range(NSUB):
            cps.append(pltpu.async_copy(
                a_h.at[idxs_v.at[j]], rowsa_v.at[pl.ds(j * SUB, SUB)], sem))
            cps.append(pltpu.async_copy(
                b_h.at[idxd_v.at[j]], rowsb_v.at[pl.ds(j * SUB, SUB)], sem))
        for cp in cps:
            cp.wait()

        def _group(g, carry2):
            cav = ca_v[pl.ds(g * 16, 16)]
            for e in range(16):
                ee = g * 16 + e
                lo = rowsa_v[ee, pl.ds(0, 16)] + rowsb_v[ee, pl.ds(0, 16)]
                hi = rowsa_v[ee, pl.ds(16, 16)] + rowsb_v[ee, pl.ds(16, 16)]
                t = hi + cav[e]
                sig = 1.0 / (1.0 + jnp.exp(-t))
                u = jnp.maximum(lo + c_v[ee, :], 0.0)
                ubf_v[ee, :] = u
                msg_v[ee, :] = u * sig[0]
            return carry2

        lax.fori_loop(0, NG, _group, 0)
        pltpu.sync_copy(ubf_v, ubf_h.at[pl.ds(ebase, CB)])
        for j in range(NSUB):
            pltpu.sync_copy(msg_v.at[pl.ds(j * SUB, SUB), :],
                            agg_sh.at[idxd_v.at[j]], add=True)
        return carry

    lax.fori_loop(0, NCH, _chunk, 0)
    plsc.subcore_barrier()

    pltpu.sync_copy(agg_sh.at[pl.ds(zbase, ZR)],
                    part_h.at[c_id, pl.ds(zbase, ZR)])

    @pl.when(s_id == 15)
    def _dtail():
        pltpu.sync_copy(agg_sh.at[pl.ds(16 * ZR, ZTAIL)],
                        part_h.at[c_id, pl.ds(16 * ZR, ZTAIL)])


def _sc_edge(a_n, b_n, src, dst, c_e, ca_e):
    mesh = plsc.VectorSubcoreMesh(core_axis_name="c", subcore_axis_name="s")
    kern = functools.partial(
        pl.kernel,
        mesh=mesh,
        compiler_params=pltpu.CompilerParams(use_tc_tiling_on_sc=False),
        out_type=[
            jax.ShapeDtypeStruct((N_EDGES, D_EDGE), jnp.float32),
            jax.ShapeDtypeStruct((2, N_NODES, D_EDGE), jnp.float32),
        ],
        scratch_types=[
            pltpu.VMEM((NSUB, SUB), jnp.int32),
            pltpu.VMEM((NSUB, SUB), jnp.int32),
            pltpu.VMEM((CB, 32), jnp.float32),
            pltpu.VMEM((CB, 32), jnp.float32),
            pltpu.VMEM((CB, D_EDGE), jnp.float32),
            pltpu.VMEM((CB,), jnp.float32),
            pltpu.VMEM((CB, D_EDGE), jnp.float32),
            pltpu.VMEM((CB, D_EDGE), jnp.float32),
            pltpu.VMEM_SHARED((N_NODES, D_EDGE), jnp.float32),
            pltpu.SemaphoreType.DMA,
        ],
    )(_sc_edge_body)
    return kern(a_n, b_n, src, dst, c_e, ca_e)


def _nodemodel_body(af_ref, p0_ref, p1_ref, wn1_ref, wn2_ref, bn_ref, out_ref):
    acc = jnp.dot(af_ref[...], wn1_ref[...], preferred_element_type=jnp.float32)
    acc = acc + jnp.dot(p0_ref[...] + p1_ref[...], wn2_ref[...],
                        preferred_element_type=jnp.float32)
    out_ref[...] = jnp.maximum(acc + bn_ref[...], 0.0)


def _node_model(af, p0, p1, wn1, wn2, bn):
    blk = 2000
    grid = (N_NODES // blk,)
    return pl.pallas_call(
        _nodemodel_body,
        grid=grid,
        in_specs=[
            pl.BlockSpec((blk, D_FEAT), lambda i: (i, 0)),
            pl.BlockSpec((blk, D_EDGE), lambda i: (i, 0)),
            pl.BlockSpec((blk, D_EDGE), lambda i: (i, 0)),
            pl.BlockSpec((D_FEAT, D_FEAT), lambda i: (0, 0)),
            pl.BlockSpec((D_EDGE, D_FEAT), lambda i: (0, 0)),
            pl.BlockSpec((1, D_FEAT), lambda i: (0, 0)),
        ],
        out_specs=pl.BlockSpec((blk, D_FEAT), lambda i: (i, 0)),
        out_shape=jax.ShapeDtypeStruct((N_NODES, D_FEAT), jnp.float32),
    )(af, p0, p1, wn1, wn2, bn)


def kernel(af, edge_index, bf, W_edge, b_edge, W_attn, b_attn, W_node, b_node):
    zpad = jnp.zeros((D_FEAT, 32 - D_EDGE - 1), jnp.float32)
    w1p = jnp.concatenate([W_edge[:D_FEAT], W_attn[:D_FEAT], zpad], axis=1)
    w2p = jnp.concatenate([W_edge[D_FEAT:2 * D_FEAT],
                           W_attn[D_FEAT:2 * D_FEAT], zpad], axis=1)
    w3 = W_edge[2 * D_FEAT:]
    w4 = W_attn[2 * D_FEAT:]

    a_n, b_n = _node_projections(af, w1p, w2p)
    c_e, ca_e = _edge_projections(bf, w3, b_edge.reshape(1, D_EDGE),
                                  w4, b_attn.reshape(1, 1))

    ubf, part = _sc_edge(a_n, b_n, edge_index[0], edge_index[1],
                         c_e, ca_e.reshape(N_EDGES))

    uaf = _node_model(af, part[0], part[1],
                      W_node[:D_FEAT], W_node[D_FEAT:],
                      b_node.reshape(1, D_FEAT))
    return (uaf, ubf)
